# baseline (speedup 1.0000x reference)
import functools

import jax
import jax.numpy as jnp
from jax import lax
from jax.experimental import pallas as pl
from jax.experimental.pallas import tpu as pltpu
from jax.experimental.pallas import tpu_sc as plsc

B = 16384
EMB = 16
NC = 2
NS = 16
L = 16
NW = NC * NS
BPW = B // NW
NCH = 4
CH = BPW // NCH
GPC = CH // L
VOCAB = 1000000


@functools.partial(
    pl.kernel,
    mesh=plsc.VectorSubcoreMesh(core_axis_name="c", subcore_axis_name="s"),
    compiler_params=pltpu.CompilerParams(
        needs_layout_passes=False, use_tc_tiling_on_sc=False
    ),
    out_type=[
        jax.ShapeDtypeStruct((NW, BPW), jnp.float32),
        jax.ShapeDtypeStruct((NW, L), jnp.float32),
    ],
    scratch_types=[
        pltpu.VMEM((NCH, CH), jnp.int32),
        pltpu.VMEM((NCH, CH), jnp.int32),
        pltpu.VMEM((NCH, CH), jnp.int32),
        pltpu.VMEM((NCH, CH), jnp.int32),
        pltpu.VMEM((BPW, EMB), jnp.float32),
        pltpu.VMEM((BPW, EMB), jnp.float32),
        pltpu.VMEM((BPW, L), jnp.float32),
        pltpu.VMEM((BPW, L), jnp.float32),
        pltpu.VMEM((BPW,), jnp.float32),
        pltpu.VMEM((L,), jnp.float32),
        pltpu.SemaphoreType.DMA,
    ],
)
def _fm_sc(users_hbm, items_hbm, uemb_hbm, iemb_hbm, ulin_hbm, ilin_hbm,
           out_hbm, aux_hbm,
           uidx_v, iidx_v, ulidx_v, ilidx_v, ue_v, ie_v, ul_v, il_v,
           out_v, acc_v, sem):
    wid = lax.axis_index("s") * NC + lax.axis_index("c")

    pltpu.sync_copy(users_hbm.at[wid], uidx_v)
    pltpu.sync_copy(items_hbm.at[wid], iidx_v)

    for ch in range(NCH):
        for k in range(GPC):
            s = pl.ds(k * L, L)
            ulidx_v[ch, s] = uidx_v[ch, s] >> 4
            ilidx_v[ch, s] = iidx_v[ch, s] >> 4

    copies = []
    for ch in range(NCH):
        rows = pl.ds(ch * CH, CH)
        copies.append(pltpu.async_copy(uemb_hbm.at[uidx_v.at[ch]], ue_v.at[rows], sem))
        copies.append(pltpu.async_copy(iemb_hbm.at[iidx_v.at[ch]], ie_v.at[rows], sem))
        copies.append(pltpu.async_copy(ulin_hbm.at[ulidx_v.at[ch]], ul_v.at[rows], sem))
        copies.append(pltpu.async_copy(ilin_hbm.at[ilidx_v.at[ch]], il_v.at[rows], sem))
    for cp in copies:
        cp.wait()

    def group(g, acc):
        rows = g * L + lax.iota(jnp.int32, L)
        fm = jnp.zeros((L,), jnp.float32)
        for d in range(EMB):
            dd = jnp.full((L,), d, jnp.int32)
            uc = plsc.load_gather(ue_v, [rows, dd])
            ic = plsc.load_gather(ie_v, [rows, dd])
            fm = fm + uc * ic
        uid = plsc.load_gather(uidx_v, [rows >> 7, rows & (CH - 1)])
        iid = plsc.load_gather(iidx_v, [rows >> 7, rows & (CH - 1)])
        ul = plsc.load_gather(ul_v, [rows, uid & (L - 1)])
        il = plsc.load_gather(il_v, [rows, iid & (L - 1)])
        x = ul + il + fm
        sig = 1.0 / (1.0 + jnp.exp(-x))
        plsc.store_scatter(out_v, [rows], sig)
        return acc + fm * fm

    acc = lax.fori_loop(0, BPW // L, group, jnp.zeros((L,), jnp.float32))
    acc_v[...] = acc

    pltpu.sync_copy(out_v, out_hbm.at[wid])
    pltpu.sync_copy(acc_v, aux_hbm.at[wid])


def kernel(users, items, user_emb, item_emb, user_lin, item_lin):
    u = users.reshape(NW, NCH, CH).astype(jnp.int32)
    i = items.reshape(NW, NCH, CH).astype(jnp.int32)
    ulin = user_lin.reshape(VOCAB // L, L)
    ilin = item_lin.reshape(VOCAB // L, L)
    sig, parts = _fm_sc(u, i, user_emb, item_emb, ulin, ilin)
    aux = 0.1 * (jnp.sum(parts) / B)
    return (sig.reshape(B, 1), aux)
